# baseline (speedup 1.0000x reference)
import functools

import jax
import jax.numpy as jnp
import numpy as np
from jax import lax
from jax.experimental import pallas as pl
from jax.experimental.pallas import tpu as pltpu
from jax.experimental.pallas import tpu_sc as plsc

N_NODES = 10000
N_EDGES = 320000
D_NODE = 128
D_EDGE = 16
D_OUT = 128

NC = 2
NS = 16
NW = NC * NS
EPW = N_EDGES // NW
EB = 40
NB = EPW // EB
N_PAD = 10240
ROWS_PER_TILE = N_PAD // NS


def _prep_body(x_ref, w1ab_ref, p_ref, q_ref):
    w1a = w1ab_ref[0:D_NODE, :]
    w1b = w1ab_ref[D_NODE:2 * D_NODE, :]
    x = x_ref[...]
    p_ref[...] = jnp.dot(x, w1a - w1b, preferred_element_type=jnp.float32)
    q_ref[...] = jnp.dot(x, w1b, preferred_element_type=jnp.float32)


def _prep(x, w1ab):
    return pl.pallas_call(
        _prep_body,
        out_shape=[
            jax.ShapeDtypeStruct((N_NODES, D_NODE), jnp.float32),
            jax.ShapeDtypeStruct((N_NODES, D_NODE), jnp.float32),
        ],
    )(x, w1ab)


EBLK = 8000


def _edge_bias_body(ea_ref, w1_ref, b1_ref, c_ref):
    w1c = w1_ref[2 * D_NODE:2 * D_NODE + D_EDGE, :]
    c_ref[...] = (
        jnp.dot(ea_ref[...], w1c, preferred_element_type=jnp.float32)
        + b1_ref[...]
    ).astype(jnp.bfloat16)


def _edge_bias(edge_attr, w1, b1_row):
    return pl.pallas_call(
        _edge_bias_body,
        grid=(N_EDGES // EBLK,),
        in_specs=[
            pl.BlockSpec((EBLK, D_EDGE), lambda i: (i, 0)),
            pl.BlockSpec((2 * D_NODE + D_EDGE, D_OUT), lambda i: (0, 0)),
            pl.BlockSpec((1, D_OUT), lambda i: (0, 0)),
        ],
        out_specs=pl.BlockSpec((EBLK, D_OUT), lambda i: (i, 0)),
        out_shape=jax.ShapeDtypeStruct((N_EDGES, D_OUT), jnp.bfloat16),
    )(edge_attr, w1, b1_row)


IB = 10
NBATCH = NB // IB


def _sc_body(p_hbm, q_hbm, c_hbm, src_hbm, dst_hbm, out_hbm, deg_hbm,
             s_sh, dflat_v, sflat_v, d0_v, s0_v, d1_v, s1_v,
             p0_v, q0_v, c0_v, p1_v, q1_v, c1_v, hist_v,
             sem_g0, sem_g1, sem_s0, sem_s1):
    c = lax.axis_index("c")
    s = lax.axis_index("s")
    wid = s * NC + c
    ebase = wid * EPW

    zvec = jnp.zeros((16,), jnp.float32)
    ones = jnp.ones((16,), jnp.float32)

    def _zero_st(i, _):
        for k in range(D_OUT // 16):
            p0_v[i, pl.ds(k * 16, 16)] = zvec
        return 0
    lax.fori_loop(0, EB, _zero_st, 0)
    for j in range(ROWS_PER_TILE // EB):
        pltpu.sync_copy(
            p0_v, s_sh.at[pl.ds(s * ROWS_PER_TILE + j * EB, EB)])

    def _zero_hist(i, _):
        hist_v[pl.ds(i * 16, 16)] = zvec
        return 0
    lax.fori_loop(0, N_PAD // 16, _zero_hist, 0)

    plsc.subcore_barrier()

    def _load_batch(t):
        off = ebase + t * (IB * EB)
        pltpu.sync_copy(dst_hbm.at[pl.ds(off, IB * EB)], dflat_v)
        pltpu.sync_copy(src_hbm.at[pl.ds(off, IB * EB)], sflat_v)
        for i in range(IB * EB // 16):
            plsc.addupdate_scatter(
                hist_v, [dflat_v[pl.ds(i * 16, 16)]], ones)

    def _issue(r, eoff, d_v, s_v, p_v, q_v, cc_v, sem):
        o = r * EB
        d_v[pl.ds(0, 16)] = dflat_v[pl.ds(o, 16)]
        d_v[pl.ds(16, 16)] = dflat_v[pl.ds(o + 16, 16)]
        d_v[pl.ds(24, 16)] = dflat_v[pl.ds(o + 24, 16)]
        s_v[pl.ds(0, 16)] = sflat_v[pl.ds(o, 16)]
        s_v[pl.ds(16, 16)] = sflat_v[pl.ds(o + 16, 16)]
        s_v[pl.ds(24, 16)] = sflat_v[pl.ds(o + 24, 16)]
        pltpu.async_copy(p_hbm.at[d_v], p_v, sem)
        pltpu.async_copy(q_hbm.at[s_v], q_v, sem)
        pltpu.async_copy(c_hbm.at[pl.ds(eoff, EB)], cc_v, sem)

    def _drain_gather(d_v, s_v, p_v, q_v, cc_v, sem):
        pltpu.make_async_copy(p_hbm.at[d_v], p_v, sem).wait()
        pltpu.make_async_copy(q_hbm.at[s_v], q_v, sem).wait()
        pltpu.make_async_copy(c_hbm.at[pl.ds(0, EB)], cc_v, sem).wait()

    def _drain_scatter(pp_v, sem):
        pltpu.make_async_copy(pp_v, s_sh.at[pl.ds(0, EB)], sem).wait()

    sh16 = jnp.full((16,), 16, jnp.int32)
    hi16 = jnp.full((16,), -65536, jnp.int32)

    def _compute(p_v, q_v, cc_v):
        def _edge(e, _):
            for k in range(D_OUT // 32):
                o = k * 32
                wc = cc_v[e, pl.ds(o // 2, 16)]
                ca = plsc.bitcast(lax.shift_left(wc, sh16), jnp.float32)
                cb = plsc.bitcast(lax.bitwise_and(wc, hi16), jnp.float32)
                va = p_v[e, pl.ds(o, 16)] + q_v[e, pl.ds(o, 16)] + ca
                vb = (p_v[e, pl.ds(o + 16, 16)]
                      + q_v[e, pl.ds(o + 16, 16)] + cb)
                p_v[e, pl.ds(o, 16)] = jnp.maximum(va, 0.0)
                p_v[e, pl.ds(o + 16, 16)] = jnp.maximum(vb, 0.0)
            return 0
        lax.fori_loop(0, EB, _edge, 0)

    _load_batch(0)
    _issue(0, ebase, d0_v, s0_v, p0_v, q0_v, c0_v, sem_g0)

    HALF = NB // 2

    def _outer(g, _):
        m = g % (IB // 2)
        eoff0 = ebase + g * (2 * EB)

        @pl.when(g > 0)
        def _():
            _drain_scatter(p1_v, sem_s1)
        _issue(2 * m + 1, eoff0 + EB, d1_v, s1_v, p1_v, q1_v, c1_v, sem_g1)

        _drain_gather(d0_v, s0_v, p0_v, q0_v, c0_v, sem_g0)
        _compute(p0_v, q0_v, c0_v)
        pltpu.async_copy(p0_v, s_sh.at[d0_v], sem_s0, add=True)

        @pl.when((m == IB // 2 - 1) & (g < HALF - 1))
        def _():
            _load_batch(g // (IB // 2) + 1)

        @pl.when(g < HALF - 1)
        def _():
            _drain_scatter(p0_v, sem_s0)
            rnext = jnp.where(m == IB // 2 - 1, 0, 2 * m + 2)
            _issue(rnext, eoff0 + 2 * EB, d0_v, s0_v, p0_v, q0_v, c0_v,
                   sem_g0)

        _drain_gather(d1_v, s1_v, p1_v, q1_v, c1_v, sem_g1)
        _compute(p1_v, q1_v, c1_v)
        pltpu.async_copy(p1_v, s_sh.at[d1_v], sem_s1, add=True)
        return 0

    lax.fori_loop(0, HALF, _outer, 0)
    _drain_scatter(p0_v, sem_s0)
    _drain_scatter(p1_v, sem_s1)

    plsc.subcore_barrier()

    pltpu.sync_copy(s_sh.at[pl.ds(s * ROWS_PER_TILE, ROWS_PER_TILE)],
                    out_hbm.at[c, pl.ds(s * ROWS_PER_TILE, ROWS_PER_TILE)])
    pltpu.sync_copy(hist_v, deg_hbm.at[c, s])


_sc_edge = pl.kernel(
    _sc_body,
    out_type=(jax.ShapeDtypeStruct((NC, N_PAD, D_OUT), jnp.float32),
              jax.ShapeDtypeStruct((NC, NS, N_PAD), jnp.float32)),
    mesh=plsc.VectorSubcoreMesh(core_axis_name="c", subcore_axis_name="s"),
    compiler_params=pltpu.CompilerParams(needs_layout_passes=False),
    scratch_types=[
        pltpu.VMEM_SHARED((N_PAD, D_OUT), jnp.float32),
        pltpu.VMEM((IB * EB,), jnp.int32),
        pltpu.VMEM((IB * EB,), jnp.int32),
        pltpu.VMEM((EB,), jnp.int32),
        pltpu.VMEM((EB,), jnp.int32),
        pltpu.VMEM((EB,), jnp.int32),
        pltpu.VMEM((EB,), jnp.int32),
        pltpu.VMEM((EB, D_NODE), jnp.float32),
        pltpu.VMEM((EB, D_NODE), jnp.float32),
        pltpu.VMEM((EB, D_OUT // 2), jnp.int32),
        pltpu.VMEM((EB, D_NODE), jnp.float32),
        pltpu.VMEM((EB, D_NODE), jnp.float32),
        pltpu.VMEM((EB, D_OUT // 2), jnp.int32),
        pltpu.VMEM((N_PAD,), jnp.float32),
        pltpu.SemaphoreType.DMA,
        pltpu.SemaphoreType.DMA,
        pltpu.SemaphoreType.DMA,
        pltpu.SemaphoreType.DMA,
    ],
)


def _final_body(sp_ref, deg_ref, w2_ref, b2_ref, gamma_ref, beta_ref,
                out_ref):
    h = sp_ref[0, :N_NODES, :] + sp_ref[1, :N_NODES, :]
    deg = jnp.sum(deg_ref[...], axis=(0, 1))[:N_NODES].reshape(N_NODES, 1)
    agg = (jnp.dot(h, w2_ref[...], preferred_element_type=jnp.float32)
           + deg * b2_ref[...])
    mean = jnp.mean(agg, axis=0, keepdims=True)
    var = jnp.mean((agg - mean) ** 2, axis=0, keepdims=True)
    out = (agg - mean) * lax.rsqrt(var + 1e-5) * gamma_ref[...] + beta_ref[...]
    out_ref[...] = jnp.maximum(out, 0.0)


def _final(spart, deg, w2, b2_row, gamma_row, beta_row):
    return pl.pallas_call(
        _final_body,
        out_shape=jax.ShapeDtypeStruct((N_NODES, D_OUT), jnp.float32),
    )(spart, deg, w2, b2_row, gamma_row, beta_row)


_PHI = np.concatenate([
    np.concatenate([np.arange(o, o + 32, 2), np.arange(o + 1, o + 32, 2)])
    for o in range(0, D_OUT, 32)
])


def kernel(x, edge_index, edge_attr, W1, b1, W2, b2, gamma, beta):
    idx = edge_index.astype(jnp.int32)
    src = idx[0]
    dst = idx[1]
    p, q = _prep(x, W1[:2 * D_NODE][:, _PHI])
    c = _edge_bias(edge_attr, W1, b1.reshape(1, D_OUT))
    c32 = lax.bitcast_convert_type(
        c.reshape(N_EDGES, D_OUT // 2, 2), jnp.int32)
    spart, deg = _sc_edge(p, q, c32, src, dst)
    return _final(spart, deg, W2[_PHI], b2.reshape(1, D_OUT),
                  gamma.reshape(1, D_OUT), beta.reshape(1, D_OUT))


# speedup vs baseline: 2.4957x; 2.4957x over previous
import functools

import jax
import jax.numpy as jnp
import numpy as np
from jax import lax
from jax.experimental import pallas as pl
from jax.experimental.pallas import tpu as pltpu
from jax.experimental.pallas import tpu_sc as plsc

N_NODES = 10000
N_EDGES = 320000
D_NODE = 128
D_EDGE = 16
D_OUT = 128

NC = 2
NS = 16
NW = NC * NS
EPW = N_EDGES // NW
EB = 40
NB = EPW // EB
N_PAD = 10240
ROWS_PER_TILE = N_PAD // NS


def _prep_body(x_ref, w1ab_ref, p_ref, q_ref):
    w1a = w1ab_ref[0:D_NODE, :]
    w1b = w1ab_ref[D_NODE:2 * D_NODE, :]
    x = x_ref[...]
    p_ref[...] = jnp.dot(x, w1a - w1b, preferred_element_type=jnp.float32)
    q_ref[...] = jnp.dot(x, w1b, preferred_element_type=jnp.float32)


def _prep(x, w1ab):
    return pl.pallas_call(
        _prep_body,
        out_shape=[
            jax.ShapeDtypeStruct((N_NODES, D_NODE), jnp.float32),
            jax.ShapeDtypeStruct((N_NODES, D_NODE), jnp.float32),
        ],
    )(x, w1ab)


EBLK = 8000


def _rne_bf16_bits(f32):
    u = lax.bitcast_convert_type(f32, jnp.uint32)
    one = jnp.uint32(1)
    sixteen = jnp.uint32(16)
    lsb = lax.bitwise_and(lax.shift_right_logical(u, sixteen), one)
    return lax.shift_right_logical(u + jnp.uint32(0x7FFF) + lsb, sixteen)


def _edge_bias_body(ea_ref, w1_ref, b1_ref, c_ref):
    w1c = w1_ref[2 * D_NODE:2 * D_NODE + D_EDGE, :]
    cf = (jnp.dot(ea_ref[...], w1c, preferred_element_type=jnp.float32)
          + b1_ref[...])
    lo = _rne_bf16_bits(cf[:, :D_OUT // 2])
    hi = lax.shift_left(_rne_bf16_bits(cf[:, D_OUT // 2:]), jnp.uint32(16))
    c_ref[...] = lax.bitcast_convert_type(hi | lo, jnp.int32)


def _edge_bias(edge_attr, w1, b1_row):
    return pl.pallas_call(
        _edge_bias_body,
        grid=(N_EDGES // EBLK,),
        in_specs=[
            pl.BlockSpec((EBLK, D_EDGE), lambda i: (i, 0)),
            pl.BlockSpec((2 * D_NODE + D_EDGE, D_OUT), lambda i: (0, 0)),
            pl.BlockSpec((1, D_OUT), lambda i: (0, 0)),
        ],
        out_specs=pl.BlockSpec((EBLK, D_OUT // 2), lambda i: (i, 0)),
        out_shape=jax.ShapeDtypeStruct((N_EDGES, D_OUT // 2), jnp.int32),
    )(edge_attr, w1, b1_row)


IB = 10
NBATCH = NB // IB


def _sc_body(p_hbm, q_hbm, c_hbm, src_hbm, dst_hbm, out_hbm, deg_hbm,
             s_sh, dflat_v, sflat_v, d0_v, s0_v, d1_v, s1_v,
             p0_v, q0_v, c0_v, p1_v, q1_v, c1_v, hist_v,
             sem_g0, sem_g1, sem_s0, sem_s1):
    c = lax.axis_index("c")
    s = lax.axis_index("s")
    wid = s * NC + c
    ebase = wid * EPW

    zvec = jnp.zeros((16,), jnp.float32)
    ones = jnp.ones((16,), jnp.float32)

    def _zero_st(i, _):
        for k in range(D_OUT // 16):
            p0_v[i, pl.ds(k * 16, 16)] = zvec
        return 0
    lax.fori_loop(0, EB, _zero_st, 0)
    for j in range(ROWS_PER_TILE // EB):
        pltpu.sync_copy(
            p0_v, s_sh.at[pl.ds(s * ROWS_PER_TILE + j * EB, EB)])

    def _zero_hist(i, _):
        hist_v[pl.ds(i * 16, 16)] = zvec
        return 0
    lax.fori_loop(0, N_PAD // 16, _zero_hist, 0)

    plsc.subcore_barrier()

    def _load_batch(t):
        off = ebase + t * (IB * EB)
        pltpu.sync_copy(dst_hbm.at[pl.ds(off, IB * EB)], dflat_v)
        pltpu.sync_copy(src_hbm.at[pl.ds(off, IB * EB)], sflat_v)
        for i in range(IB * EB // 16):
            plsc.addupdate_scatter(
                hist_v, [dflat_v[pl.ds(i * 16, 16)]], ones)

    def _issue(r, eoff, d_v, s_v, p_v, q_v, cc_v, sem):
        o = r * EB
        d_v[pl.ds(0, 16)] = dflat_v[pl.ds(o, 16)]
        d_v[pl.ds(16, 16)] = dflat_v[pl.ds(o + 16, 16)]
        d_v[pl.ds(24, 16)] = dflat_v[pl.ds(o + 24, 16)]
        s_v[pl.ds(0, 16)] = sflat_v[pl.ds(o, 16)]
        s_v[pl.ds(16, 16)] = sflat_v[pl.ds(o + 16, 16)]
        s_v[pl.ds(24, 16)] = sflat_v[pl.ds(o + 24, 16)]
        pltpu.async_copy(p_hbm.at[d_v], p_v, sem)
        pltpu.async_copy(q_hbm.at[s_v], q_v, sem)
        pltpu.async_copy(c_hbm.at[pl.ds(eoff, EB)], cc_v, sem)

    def _drain_gather(d_v, s_v, p_v, q_v, cc_v, sem):
        pltpu.make_async_copy(p_hbm.at[d_v], p_v, sem).wait()
        pltpu.make_async_copy(q_hbm.at[s_v], q_v, sem).wait()
        pltpu.make_async_copy(c_hbm.at[pl.ds(0, EB)], cc_v, sem).wait()

    def _drain_scatter(pp_v, sem):
        pltpu.make_async_copy(pp_v, s_sh.at[pl.ds(0, EB)], sem).wait()

    sh16 = jnp.full((16,), 16, jnp.int32)
    hi16 = jnp.full((16,), -65536, jnp.int32)

    def _compute(p_v, q_v, cc_v):
        def _edge(e, _):
            for k in range(D_OUT // 32):
                o = k * 16
                wc = cc_v[e, pl.ds(o, 16)]
                ca = plsc.bitcast(lax.shift_left(wc, sh16), jnp.float32)
                cb = plsc.bitcast(lax.bitwise_and(wc, hi16), jnp.float32)
                va = p_v[e, pl.ds(o, 16)] + q_v[e, pl.ds(o, 16)] + ca
                vb = (p_v[e, pl.ds(o + 64, 16)]
                      + q_v[e, pl.ds(o + 64, 16)] + cb)
                p_v[e, pl.ds(o, 16)] = jnp.maximum(va, 0.0)
                p_v[e, pl.ds(o + 64, 16)] = jnp.maximum(vb, 0.0)
            return 0
        lax.fori_loop(0, EB, _edge, 0)

    _load_batch(0)
    _issue(0, ebase, d0_v, s0_v, p0_v, q0_v, c0_v, sem_g0)

    HALF = NB // 2

    def _outer(g, _):
        m = g % (IB // 2)
        eoff0 = ebase + g * (2 * EB)

        @pl.when(g > 0)
        def _():
            _drain_scatter(p1_v, sem_s1)
        _issue(2 * m + 1, eoff0 + EB, d1_v, s1_v, p1_v, q1_v, c1_v, sem_g1)

        _drain_gather(d0_v, s0_v, p0_v, q0_v, c0_v, sem_g0)
        _compute(p0_v, q0_v, c0_v)
        pltpu.async_copy(p0_v, s_sh.at[d0_v], sem_s0, add=True)

        @pl.when((m == IB // 2 - 1) & (g < HALF - 1))
        def _():
            _load_batch(g // (IB // 2) + 1)

        @pl.when(g < HALF - 1)
        def _():
            _drain_scatter(p0_v, sem_s0)
            rnext = jnp.where(m == IB // 2 - 1, 0, 2 * m + 2)
            _issue(rnext, eoff0 + 2 * EB, d0_v, s0_v, p0_v, q0_v, c0_v,
                   sem_g0)

        _drain_gather(d1_v, s1_v, p1_v, q1_v, c1_v, sem_g1)
        _compute(p1_v, q1_v, c1_v)
        pltpu.async_copy(p1_v, s_sh.at[d1_v], sem_s1, add=True)
        return 0

    lax.fori_loop(0, HALF, _outer, 0)
    _drain_scatter(p0_v, sem_s0)
    _drain_scatter(p1_v, sem_s1)

    plsc.subcore_barrier()

    pltpu.sync_copy(s_sh.at[pl.ds(s * ROWS_PER_TILE, ROWS_PER_TILE)],
                    out_hbm.at[c, pl.ds(s * ROWS_PER_TILE, ROWS_PER_TILE)])
    pltpu.sync_copy(hist_v, deg_hbm.at[c, s])


_sc_edge = pl.kernel(
    _sc_body,
    out_type=(jax.ShapeDtypeStruct((NC, N_PAD, D_OUT), jnp.float32),
              jax.ShapeDtypeStruct((NC, NS, N_PAD), jnp.float32)),
    mesh=plsc.VectorSubcoreMesh(core_axis_name="c", subcore_axis_name="s"),
    compiler_params=pltpu.CompilerParams(needs_layout_passes=False),
    scratch_types=[
        pltpu.VMEM_SHARED((N_PAD, D_OUT), jnp.float32),
        pltpu.VMEM((IB * EB,), jnp.int32),
        pltpu.VMEM((IB * EB,), jnp.int32),
        pltpu.VMEM((EB,), jnp.int32),
        pltpu.VMEM((EB,), jnp.int32),
        pltpu.VMEM((EB,), jnp.int32),
        pltpu.VMEM((EB,), jnp.int32),
        pltpu.VMEM((EB, D_NODE), jnp.float32),
        pltpu.VMEM((EB, D_NODE), jnp.float32),
        pltpu.VMEM((EB, D_OUT // 2), jnp.int32),
        pltpu.VMEM((EB, D_NODE), jnp.float32),
        pltpu.VMEM((EB, D_NODE), jnp.float32),
        pltpu.VMEM((EB, D_OUT // 2), jnp.int32),
        pltpu.VMEM((N_PAD,), jnp.float32),
        pltpu.SemaphoreType.DMA,
        pltpu.SemaphoreType.DMA,
        pltpu.SemaphoreType.DMA,
        pltpu.SemaphoreType.DMA,
    ],
)


def _final_body(sp_ref, deg_ref, w2_ref, b2_ref, gamma_ref, beta_ref,
                out_ref):
    h = sp_ref[0, :N_NODES, :] + sp_ref[1, :N_NODES, :]
    deg = jnp.sum(deg_ref[...], axis=(0, 1))[:N_NODES].reshape(N_NODES, 1)
    agg = (jnp.dot(h, w2_ref[...], preferred_element_type=jnp.float32)
           + deg * b2_ref[...])
    mean = jnp.mean(agg, axis=0, keepdims=True)
    var = jnp.mean((agg - mean) ** 2, axis=0, keepdims=True)
    out = (agg - mean) * lax.rsqrt(var + 1e-5) * gamma_ref[...] + beta_ref[...]
    out_ref[...] = jnp.maximum(out, 0.0)


def _final(spart, deg, w2, b2_row, gamma_row, beta_row):
    return pl.pallas_call(
        _final_body,
        out_shape=jax.ShapeDtypeStruct((N_NODES, D_OUT), jnp.float32),
    )(spart, deg, w2, b2_row, gamma_row, beta_row)


def kernel(x, edge_index, edge_attr, W1, b1, W2, b2, gamma, beta):
    idx = edge_index.astype(jnp.int32)
    src = idx[0]
    dst = idx[1]
    p, q = _prep(x, W1[:2 * D_NODE])
    c32 = _edge_bias(edge_attr, W1, b1.reshape(1, D_OUT))
    spart, deg = _sc_edge(p, q, c32, src, dst)
    return _final(spart, deg, W2, b2.reshape(1, D_OUT),
                  gamma.reshape(1, D_OUT), beta.reshape(1, D_OUT))
